# Initial kernel scaffold; baseline (speedup 1.0000x reference)
import jax
import jax.numpy as jnp
from jax.experimental import pallas as pl


def kernel(x, edge_index, W1, b1, W2, b2):
    raise NotImplementedError("write your pallas kernel here")


# speedup vs baseline: 8.4962x; 8.4962x over previous
import functools

import jax
import jax.numpy as jnp
from jax import lax
from jax.experimental import pallas as pl
from jax.experimental.pallas import tpu as pltpu
from jax.experimental.pallas import tpu_sc as plsc

NC = 2
NS = 16
NW = NC * NS
CHUNK = 128
DEG_W = 8
N_PAD = 10240
ROW_BLK = 1000


def _sc_degree(n_pad, g):
    rpt = n_pad // NS
    nfill = rpt // CHUNK
    mesh = plsc.VectorSubcoreMesh(
        core_axis_name="c", subcore_axis_name="s",
        num_cores=NC, num_subcores=NS)

    def body(dsts, out, dbuf, rows, acc, sem):
        c = lax.axis_index("c")
        s = lax.axis_index("s")
        wid = s * NC + c
        base = s * rpt

        def fill(i, val):
            for j in range(DEG_W // 16):
                rows[i, pl.ds(j * 16, 16)] = jnp.full((16,), val, jnp.float32)
            return val

        lax.fori_loop(0, CHUNK, fill, 0.0)
        for k in range(nfill):
            pltpu.sync_copy(rows, acc.at[pl.ds(base + k * CHUNK, CHUNK)])
        lax.fori_loop(0, CHUNK, fill, 1.0)
        plsc.subcore_barrier()

        def step(gi, carry):
            pltpu.sync_copy(dsts.at[wid, gi], dbuf.at[0])
            pltpu.sync_copy(rows, acc.at[dbuf.at[0]], add=True)
            return carry

        lax.fori_loop(0, g, step, 0)
        plsc.subcore_barrier()
        for k in range(nfill):
            r0 = base + k * CHUNK
            pltpu.sync_copy(acc.at[pl.ds(r0, CHUNK)], rows)
            pltpu.sync_copy(rows, out.at[c, pl.ds(r0, CHUNK)])

    return pl.kernel(
        body,
        out_type=jax.ShapeDtypeStruct((NC, n_pad, DEG_W), jnp.float32),
        mesh=mesh,
        scratch_types=[
            pltpu.VMEM((1, CHUNK), jnp.int32),
            pltpu.VMEM((CHUNK, DEG_W), jnp.float32),
            pltpu.VMEM_SHARED((n_pad, DEG_W), jnp.float32),
            pltpu.SemaphoreType.DMA,
        ],
    )


def _sc_scatter(n_pad, d, g):
    rpt = n_pad // NS
    nfill = rpt // CHUNK
    assert g % 2 == 0
    mesh = plsc.VectorSubcoreMesh(
        core_axis_name="c", subcore_axis_name="s",
        num_cores=NC, num_subcores=NS)

    def body(table, srcs, dsts, out, sidx, dbuf, rows_a, rows_b, acc,
             sem_a, sem_b):
        c = lax.axis_index("c")
        s = lax.axis_index("s")
        wid = s * NC + c
        base = s * rpt

        def fill(i, val):
            for j in range(d // 16):
                rows_a[i, pl.ds(j * 16, 16)] = jnp.full((16,), val, jnp.float32)
            return val

        lax.fori_loop(0, CHUNK, fill, 0.0)
        for k in range(nfill):
            pltpu.sync_copy(rows_a, acc.at[pl.ds(base + k * CHUNK, CHUNK)])
        plsc.subcore_barrier()

        pltpu.sync_copy(srcs.at[wid], sidx)
        pltpu.async_copy(
            table.at[sidx.at[pl.ds(0, CHUNK)]], rows_a, sem_a).wait()

        def half(g_this, g_next, rows_this, rows_next, sem_this, sem_next):
            cp = pltpu.async_copy(
                table.at[sidx.at[pl.ds(g_next * CHUNK, CHUNK)]],
                rows_next, sem_next)
            pltpu.sync_copy(dsts.at[wid, g_this], dbuf.at[0])
            pltpu.sync_copy(rows_this, acc.at[dbuf.at[0]], add=True)
            cp.wait()

        def step(i, carry):
            g0 = 2 * i
            half(g0, g0 + 1, rows_a, rows_b, sem_a, sem_b)
            half(g0 + 1, g0 + 2, rows_b, rows_a, sem_b, sem_a)
            return carry

        lax.fori_loop(0, g // 2, step, 0)
        plsc.subcore_barrier()
        for k in range(nfill):
            r0 = base + k * CHUNK
            pltpu.sync_copy(acc.at[pl.ds(r0, CHUNK)], rows_a)
            pltpu.sync_copy(rows_a, out.at[c, pl.ds(r0, CHUNK)])

    return pl.kernel(
        body,
        out_type=jax.ShapeDtypeStruct((NC, n_pad, d), jnp.float32),
        mesh=mesh,
        compiler_params=pltpu.CompilerParams(
            use_tc_tiling_on_sc=(d % 128 == 0)),
        scratch_types=[
            pltpu.VMEM(((g + 1) * CHUNK,), jnp.int32),
            pltpu.VMEM((1, CHUNK), jnp.int32),
            pltpu.VMEM((CHUNK, d), jnp.float32),
            pltpu.VMEM((CHUNK, d), jnp.float32),
            pltpu.VMEM_SHARED((n_pad, d), jnp.float32),
            pltpu.SemaphoreType.DMA,
            pltpu.SemaphoreType.DMA,
        ],
    )


def _deg_rsqrt(dp0, dp1):
    deg = dp0[:, 0:1] + dp1[:, 0:1] + 1.0
    return lax.rsqrt(deg)


def _tc1_body(dp0, dp1, x, w, o):
    d = _deg_rsqrt(dp0[...], dp1[...])
    o[...] = d * jnp.dot(x[...], w[...], preferred_element_type=jnp.float32)


def _tc2_body(dp0, dp1, p0, p1, yd, b, w, o):
    d = _deg_rsqrt(dp0[...], dp1[...])
    h = jnp.maximum(d * (p0[...] + p1[...] + yd[...]) + b[...], 0.0)
    o[...] = d * jnp.dot(h, w[...], preferred_element_type=jnp.float32)


def _tc3_body(dp0, dp1, q0, q1, yd, b, o):
    d = _deg_rsqrt(dp0[...], dp1[...])
    z = d * (q0[...] + q1[...] + yd[...]) + b[...]
    m = jnp.max(z, axis=1, keepdims=True)
    e = jnp.exp(z - m)
    o[...] = (z - m) - jnp.log(jnp.sum(e, axis=1, keepdims=True))


def _row_spec(width):
    return pl.BlockSpec((ROW_BLK, width), lambda i: (i, 0))


def _full_spec(shape):
    return pl.BlockSpec(shape, lambda i: tuple(0 for _ in shape))


def kernel(x, edge_index, W1, b1, W2, b2):
    n, d_in = x.shape
    d_hid = W1.shape[1]
    d_out = W2.shape[1]
    e = edge_index.shape[1]
    grid = (n // ROW_BLK,)

    g = -(-e // (NW * CHUNK))
    g += g % 2
    e_pad = NW * g * CHUNK

    ei = edge_index.astype(jnp.int32)
    src = jnp.full((e_pad,), 0, jnp.int32).at[:e].set(ei[0])
    dst = jnp.full((e_pad,), n, jnp.int32).at[:e].set(ei[1])
    src = src.reshape(NW, g, CHUNK)
    src = jnp.concatenate([src, jnp.zeros((NW, 1, CHUNK), jnp.int32)], axis=1)
    src = src.reshape(NW, (g + 1) * CHUNK)
    dst = dst.reshape(NW, g, CHUNK)

    degp = _sc_degree(N_PAD, g)(dst)
    dp0 = degp[0, :n, :]
    dp1 = degp[1, :n, :]

    yd1 = pl.pallas_call(
        _tc1_body,
        grid=grid,
        in_specs=[_row_spec(DEG_W), _row_spec(DEG_W),
                  _row_spec(d_in), _full_spec((d_in, d_hid))],
        out_specs=_row_spec(d_hid),
        out_shape=jax.ShapeDtypeStruct((n, d_hid), jnp.float32),
    )(dp0, dp1, x, W1)

    p = _sc_scatter(N_PAD, d_hid, g)(yd1, src, dst)

    yd2 = pl.pallas_call(
        _tc2_body,
        grid=grid,
        in_specs=[_row_spec(DEG_W), _row_spec(DEG_W),
                  _row_spec(d_hid), _row_spec(d_hid), _row_spec(d_hid),
                  _full_spec((1, d_hid)), _full_spec((d_hid, d_out))],
        out_specs=_row_spec(d_out),
        out_shape=jax.ShapeDtypeStruct((n, d_out), jnp.float32),
    )(dp0, dp1, p[0, :n, :], p[1, :n, :], yd1, b1.reshape(1, d_hid), W2)

    q = _sc_scatter(N_PAD, d_out, g)(yd2, src, dst)

    out = pl.pallas_call(
        _tc3_body,
        grid=grid,
        in_specs=[_row_spec(DEG_W), _row_spec(DEG_W),
                  _row_spec(d_out), _row_spec(d_out), _row_spec(d_out),
                  _full_spec((1, d_out))],
        out_specs=_row_spec(d_out),
        out_shape=jax.ShapeDtypeStruct((n, d_out), jnp.float32),
    )(dp0, dp1, q[0, :n, :], q[1, :n, :], yd2, b2.reshape(1, d_out))

    return out


# speedup vs baseline: 16.5229x; 1.9448x over previous
import functools

import jax
import jax.numpy as jnp
from jax import lax
from jax.experimental import pallas as pl
from jax.experimental.pallas import tpu as pltpu
from jax.experimental.pallas import tpu_sc as plsc

NC = 2
NS = 16
NW = NC * NS
CHUNK = 128
DEG_W = 8
N_PAD = 10240
ROW_BLK = 1000


def _sc_degree(n_pad, g):
    rpt = n_pad // NS
    nfill = rpt // CHUNK
    mesh = plsc.VectorSubcoreMesh(
        core_axis_name="c", subcore_axis_name="s",
        num_cores=NC, num_subcores=NS)

    def body(dsts, out, dbuf, rows, acc, sem):
        c = lax.axis_index("c")
        s = lax.axis_index("s")
        wid = s * NC + c
        base = s * rpt

        def fill(i, val):
            for j in range(DEG_W // 16):
                rows[i, pl.ds(j * 16, 16)] = jnp.full((16,), val, jnp.float32)
            return val

        lax.fori_loop(0, CHUNK, fill, 0.0)
        for k in range(nfill):
            pltpu.sync_copy(rows, acc.at[pl.ds(base + k * CHUNK, CHUNK)])
        lax.fori_loop(0, CHUNK, fill, 1.0)
        plsc.subcore_barrier()

        def step(gi, carry):
            pltpu.sync_copy(dsts.at[wid, gi], dbuf.at[0])
            pltpu.sync_copy(rows, acc.at[dbuf.at[0]], add=True)
            return carry

        lax.fori_loop(0, g, step, 0)
        plsc.subcore_barrier()
        for k in range(nfill):
            r0 = base + k * CHUNK
            pltpu.sync_copy(acc.at[pl.ds(r0, CHUNK)], rows)
            pltpu.sync_copy(rows, out.at[c, pl.ds(r0, CHUNK)])

    return pl.kernel(
        body,
        out_type=jax.ShapeDtypeStruct((NC, n_pad, DEG_W), jnp.float32),
        mesh=mesh,
        scratch_types=[
            pltpu.VMEM((1, CHUNK), jnp.int32),
            pltpu.VMEM((CHUNK, DEG_W), jnp.float32),
            pltpu.VMEM_SHARED((n_pad, DEG_W), jnp.float32),
            pltpu.SemaphoreType.DMA,
        ],
    )


def _sc_scatter(n_pad, dh, g):
    rpt = n_pad // NS
    nfill = rpt // CHUNK
    assert g % 2 == 0
    mesh = plsc.VectorSubcoreMesh(
        core_axis_name="c", subcore_axis_name="s",
        num_cores=NC, num_subcores=NS)

    def body(t0, t1, idx, out, iba, ibb, rows_a, rows_b, tsp, acc,
             sem_ia, sem_ib, sem_a, sem_b):
        c = lax.axis_index("c")
        s = lax.axis_index("s")
        base = s * rpt

        def fill(i, val):
            for j in range(dh // 16):
                rows_a[i, pl.ds(j * 16, 16)] = jnp.full((16,), val, jnp.float32)
            return val

        lax.fori_loop(0, CHUNK, fill, 0.0)
        for k in range(nfill):
            pltpu.sync_copy(rows_a, acc.at[pl.ds(base + k * CHUNK, CHUNK)])

        @pl.when(c == 0)
        def _():
            pltpu.sync_copy(t0.at[pl.ds(base, rpt)], tsp.at[pl.ds(base, rpt)])

        @pl.when(c == 1)
        def _():
            pltpu.sync_copy(t1.at[pl.ds(base, rpt)], tsp.at[pl.ds(base, rpt)])

        plsc.subcore_barrier()

        def step(gi, carry):
            pltpu.sync_copy(idx.at[s, gi], iba)
            pltpu.async_copy(tsp.at[iba.at[0]], rows_a, sem_a).wait()
            pltpu.sync_copy(rows_a, acc.at[iba.at[1]], add=True)
            return carry

        lax.fori_loop(0, g, step, 0)
        plsc.subcore_barrier()
        for k in range(nfill):
            r0 = base + k * CHUNK
            pltpu.sync_copy(acc.at[pl.ds(r0, CHUNK)], rows_a)
            pltpu.sync_copy(rows_a, out.at[c, pl.ds(r0, CHUNK)])

    return pl.kernel(
        body,
        out_type=jax.ShapeDtypeStruct((NC, n_pad, dh), jnp.float32),
        mesh=mesh,
        compiler_params=pltpu.CompilerParams(use_tc_tiling_on_sc=False),
        scratch_types=[
            pltpu.VMEM((2, CHUNK), jnp.int32),
            pltpu.VMEM((2, CHUNK), jnp.int32),
            pltpu.VMEM((CHUNK, dh), jnp.float32),
            pltpu.VMEM((CHUNK, dh), jnp.float32),
            pltpu.VMEM_SHARED((n_pad, dh), jnp.float32),
            pltpu.VMEM_SHARED((n_pad, dh), jnp.float32),
            pltpu.SemaphoreType.DMA,
            pltpu.SemaphoreType.DMA,
            pltpu.SemaphoreType.DMA,
            pltpu.SemaphoreType.DMA,
        ],
    )


def _deg_rsqrt(dp0, dp1):
    deg = dp0[:, 0:1] + dp1[:, 0:1] + 1.0
    return lax.rsqrt(deg)


def _tc1_body(dp0, dp1, x, w, o0, o1):
    d = _deg_rsqrt(dp0[...], dp1[...])
    y = d * jnp.dot(x[...], w[...], preferred_element_type=jnp.float32)
    dh = y.shape[1] // 2
    o0[...] = y[:, :dh]
    o1[...] = y[:, dh:]


def _tc2_body(dp0, dp1, p0, p1, y0, y1, b, w, o0, o1):
    d = _deg_rsqrt(dp0[...], dp1[...])
    s = jnp.concatenate([p0[...] + y0[...], p1[...] + y1[...]], axis=1)
    h = jnp.maximum(d * s + b[...], 0.0)
    y = d * jnp.dot(h, w[...], preferred_element_type=jnp.float32)
    dh = y.shape[1] // 2
    o0[...] = y[:, :dh]
    o1[...] = y[:, dh:]


def _tc3_body(dp0, dp1, q0, q1, y0, y1, b, o):
    d = _deg_rsqrt(dp0[...], dp1[...])
    z = jnp.concatenate([q0[...] + y0[...], q1[...] + y1[...]], axis=1)
    z = d * z + b[...]
    m = jnp.max(z, axis=1, keepdims=True)
    e = jnp.exp(z - m)
    o[...] = (z - m) - jnp.log(jnp.sum(e, axis=1, keepdims=True))


def _row_spec(width):
    return pl.BlockSpec((ROW_BLK, width), lambda i: (i, 0))


def _full_spec(shape):
    return pl.BlockSpec(shape, lambda i: tuple(0 for _ in shape))


def kernel(x, edge_index, W1, b1, W2, b2):
    n, d_in = x.shape
    d_hid = W1.shape[1]
    d_out = W2.shape[1]
    e = edge_index.shape[1]
    grid = (n // ROW_BLK,)

    ei = edge_index.astype(jnp.int32)

    gd = -(-e // (NW * CHUNK))
    dstd = jnp.full((NW * gd * CHUNK,), n, jnp.int32).at[:e].set(ei[1])
    dstd = dstd.reshape(NW, gd, CHUNK)

    gs = -(-e // (NS * CHUNK))
    gs += gs % 2
    es = NS * gs * CHUNK
    src = jnp.full((es,), 0, jnp.int32).at[:e].set(ei[0]).reshape(NS, gs, CHUNK)
    dst = jnp.full((es,), n, jnp.int32).at[:e].set(ei[1]).reshape(NS, gs, CHUNK)
    idx = jnp.stack([src, dst], axis=2)
    pad = jnp.concatenate(
        [jnp.zeros((NS, 2, 1, CHUNK), jnp.int32),
         jnp.full((NS, 2, 1, CHUNK), n, jnp.int32)], axis=2)
    idx = jnp.concatenate([idx, pad], axis=1)

    dhid = d_hid // 2
    dout = d_out // 2

    degp = _sc_degree(N_PAD, gd)(dstd)
    dp0 = degp[0, :n, :]
    dp1 = degp[1, :n, :]

    y10, y11 = pl.pallas_call(
        _tc1_body,
        grid=grid,
        in_specs=[_row_spec(DEG_W), _row_spec(DEG_W),
                  _row_spec(d_in), _full_spec((d_in, d_hid))],
        out_specs=[_row_spec(dhid), _row_spec(dhid)],
        out_shape=[jax.ShapeDtypeStruct((N_PAD, dhid), jnp.float32),
                   jax.ShapeDtypeStruct((N_PAD, dhid), jnp.float32)],
    )(dp0, dp1, x, W1)

    p = _sc_scatter(N_PAD, dhid, gs)(y10, y11, idx)

    y20, y21 = pl.pallas_call(
        _tc2_body,
        grid=grid,
        in_specs=[_row_spec(DEG_W), _row_spec(DEG_W),
                  _row_spec(dhid), _row_spec(dhid),
                  _row_spec(dhid), _row_spec(dhid),
                  _full_spec((1, d_hid)), _full_spec((d_hid, d_out))],
        out_specs=[_row_spec(dout), _row_spec(dout)],
        out_shape=[jax.ShapeDtypeStruct((N_PAD, dout), jnp.float32),
                   jax.ShapeDtypeStruct((N_PAD, dout), jnp.float32)],
    )(dp0, dp1, p[0, :n, :], p[1, :n, :], y10[:n], y11[:n],
      b1.reshape(1, d_hid), W2)

    q = _sc_scatter(N_PAD, dout, gs)(y20, y21, idx)

    out = pl.pallas_call(
        _tc3_body,
        grid=grid,
        in_specs=[_row_spec(DEG_W), _row_spec(DEG_W),
                  _row_spec(dout), _row_spec(dout),
                  _row_spec(dout), _row_spec(dout),
                  _full_spec((1, d_out))],
        out_specs=_row_spec(d_out),
        out_shape=jax.ShapeDtypeStruct((n, d_out), jnp.float32),
    )(dp0, dp1, q[0, :n, :], q[1, :n, :], y20[:n], y21[:n],
      b2.reshape(1, d_out))

    return out


# speedup vs baseline: 18.7803x; 1.1366x over previous
import functools

import jax
import jax.numpy as jnp
from jax import lax
from jax.experimental import pallas as pl
from jax.experimental.pallas import tpu as pltpu
from jax.experimental.pallas import tpu_sc as plsc

NC = 2
NS = 16
NW = NC * NS
CHUNK = 128
DEG_W = 8
N_PAD = 10240
ROW_BLK = 1000


def _sc_degree(n_pad, g):
    rpt = n_pad // NS
    nfill = rpt // CHUNK
    mesh = plsc.VectorSubcoreMesh(
        core_axis_name="c", subcore_axis_name="s",
        num_cores=NC, num_subcores=NS)

    def body(dsts, out, dbuf, rows, acc, sem):
        c = lax.axis_index("c")
        s = lax.axis_index("s")
        wid = s * NC + c
        base = s * rpt

        def fill(i, val):
            for j in range(DEG_W // 16):
                rows[i, pl.ds(j * 16, 16)] = jnp.full((16,), val, jnp.float32)
            return val

        lax.fori_loop(0, CHUNK, fill, 0.0)
        for k in range(nfill):
            pltpu.sync_copy(rows, acc.at[pl.ds(base + k * CHUNK, CHUNK)])
        lax.fori_loop(0, CHUNK, fill, 1.0)
        plsc.subcore_barrier()

        def step(gi, carry):
            pltpu.sync_copy(dsts.at[wid, gi], dbuf.at[0])
            pltpu.sync_copy(rows, acc.at[dbuf.at[0]], add=True)
            return carry

        lax.fori_loop(0, g, step, 0)
        plsc.subcore_barrier()
        for k in range(nfill):
            r0 = base + k * CHUNK
            pltpu.sync_copy(acc.at[pl.ds(r0, CHUNK)], rows)
            pltpu.sync_copy(rows, out.at[c, pl.ds(r0, CHUNK)])

    return pl.kernel(
        body,
        out_type=jax.ShapeDtypeStruct((NC, n_pad, DEG_W), jnp.float32),
        mesh=mesh,
        scratch_types=[
            pltpu.VMEM((1, CHUNK), jnp.int32),
            pltpu.VMEM((CHUNK, DEG_W), jnp.float32),
            pltpu.VMEM_SHARED((n_pad, DEG_W), jnp.float32),
            pltpu.SemaphoreType.DMA,
        ],
    )


def _sc_scatter(n_pad, dh, g):
    rpt = n_pad // NS
    nfill = rpt // CHUNK
    assert g % 2 == 0
    mesh = plsc.VectorSubcoreMesh(
        core_axis_name="c", subcore_axis_name="s",
        num_cores=NC, num_subcores=NS)

    def body(t0, t1, idx, out, iba, ibb, rows_a, rows_b, tsp, acc,
             sem_ia, sem_ib, sem_a, sem_b):
        c = lax.axis_index("c")
        s = lax.axis_index("s")
        base = s * rpt

        def fill(i, val):
            for j in range(dh // 16):
                rows_a[i, pl.ds(j * 16, 16)] = jnp.full((16,), val, jnp.float32)
            return val

        lax.fori_loop(0, CHUNK, fill, 0.0)
        for k in range(nfill):
            pltpu.sync_copy(rows_a, acc.at[pl.ds(base + k * CHUNK, CHUNK)])

        @pl.when(c == 0)
        def _():
            pltpu.sync_copy(t0.at[pl.ds(base, rpt)], tsp.at[pl.ds(base, rpt)])

        @pl.when(c == 1)
        def _():
            pltpu.sync_copy(t1.at[pl.ds(base, rpt)], tsp.at[pl.ds(base, rpt)])

        plsc.subcore_barrier()

        pltpu.sync_copy(idx.at[s, 0], iba)
        pltpu.async_copy(tsp.at[iba.at[0]], rows_a, sem_a)

        def half(g0, ib_this, ib_next, rows_this, rows_next,
                 sem_this, sem_next):
            pltpu.sync_copy(idx.at[s, g0 + 1], ib_next)
            pltpu.async_copy(tsp.at[ib_next.at[0]], rows_next, sem_next)
            pltpu.make_async_copy(
                tsp.at[ib_this.at[0]], rows_this, sem_this).wait()
            pltpu.sync_copy(rows_this, acc.at[ib_this.at[1]], add=True)

        def step(i, carry):
            g0 = 2 * i
            half(g0, iba, ibb, rows_a, rows_b, sem_a, sem_b)
            half(g0 + 1, ibb, iba, rows_b, rows_a, sem_b, sem_a)
            return carry

        lax.fori_loop(0, g // 2, step, 0)
        pltpu.make_async_copy(tsp.at[iba.at[0]], rows_a, sem_a).wait()
        plsc.subcore_barrier()
        for k in range(nfill):
            r0 = base + k * CHUNK
            pltpu.sync_copy(acc.at[pl.ds(r0, CHUNK)], rows_a)
            pltpu.sync_copy(rows_a, out.at[c, pl.ds(r0, CHUNK)])

    return pl.kernel(
        body,
        out_type=jax.ShapeDtypeStruct((NC, n_pad, dh), jnp.float32),
        mesh=mesh,
        compiler_params=pltpu.CompilerParams(use_tc_tiling_on_sc=False),
        scratch_types=[
            pltpu.VMEM((2, CHUNK), jnp.int32),
            pltpu.VMEM((2, CHUNK), jnp.int32),
            pltpu.VMEM((CHUNK, dh), jnp.float32),
            pltpu.VMEM((CHUNK, dh), jnp.float32),
            pltpu.VMEM_SHARED((n_pad, dh), jnp.float32),
            pltpu.VMEM_SHARED((n_pad, dh), jnp.float32),
            pltpu.SemaphoreType.DMA,
            pltpu.SemaphoreType.DMA,
            pltpu.SemaphoreType.DMA,
            pltpu.SemaphoreType.DMA,
        ],
    )


def _deg_rsqrt(dp0, dp1):
    deg = dp0[:, 0:1] + dp1[:, 0:1] + 1.0
    return lax.rsqrt(deg)


def _tc1_body(dp0, dp1, x, w, o0, o1):
    d = _deg_rsqrt(dp0[...], dp1[...])
    y = d * jnp.dot(x[...], w[...], preferred_element_type=jnp.float32)
    dh = y.shape[1] // 2
    o0[...] = y[:, :dh]
    o1[...] = y[:, dh:]


def _tc2_body(dp0, dp1, p0, p1, y0, y1, b, w, o0, o1):
    d = _deg_rsqrt(dp0[...], dp1[...])
    s = jnp.concatenate([p0[...] + y0[...], p1[...] + y1[...]], axis=1)
    h = jnp.maximum(d * s + b[...], 0.0)
    y = d * jnp.dot(h, w[...], preferred_element_type=jnp.float32)
    dh = y.shape[1] // 2
    o0[...] = y[:, :dh]
    o1[...] = y[:, dh:]


def _tc3_body(dp0, dp1, q0, q1, y0, y1, b, o):
    d = _deg_rsqrt(dp0[...], dp1[...])
    z = jnp.concatenate([q0[...] + y0[...], q1[...] + y1[...]], axis=1)
    z = d * z + b[...]
    m = jnp.max(z, axis=1, keepdims=True)
    e = jnp.exp(z - m)
    o[...] = (z - m) - jnp.log(jnp.sum(e, axis=1, keepdims=True))


def _row_spec(width):
    return pl.BlockSpec((ROW_BLK, width), lambda i: (i, 0))


def _full_spec(shape):
    return pl.BlockSpec(shape, lambda i: tuple(0 for _ in shape))


def kernel(x, edge_index, W1, b1, W2, b2):
    n, d_in = x.shape
    d_hid = W1.shape[1]
    d_out = W2.shape[1]
    e = edge_index.shape[1]
    grid = (n // ROW_BLK,)

    ei = edge_index.astype(jnp.int32)

    gd = -(-e // (NW * CHUNK))
    dstd = jnp.full((NW * gd * CHUNK,), n, jnp.int32).at[:e].set(ei[1])
    dstd = dstd.reshape(NW, gd, CHUNK)

    gs = -(-e // (NS * CHUNK))
    gs += gs % 2
    es = NS * gs * CHUNK
    src = jnp.full((es,), 0, jnp.int32).at[:e].set(ei[0]).reshape(NS, gs, CHUNK)
    dst = jnp.full((es,), n, jnp.int32).at[:e].set(ei[1]).reshape(NS, gs, CHUNK)
    idx = jnp.stack([src, dst], axis=2)
    pad = jnp.concatenate(
        [jnp.zeros((NS, 2, 1, CHUNK), jnp.int32),
         jnp.full((NS, 2, 1, CHUNK), n, jnp.int32)], axis=2)
    idx = jnp.concatenate([idx, pad], axis=1)

    dhid = d_hid // 2
    dout = d_out // 2

    degp = _sc_degree(N_PAD, gd)(dstd)
    dp0 = degp[0, :n, :]
    dp1 = degp[1, :n, :]

    y10, y11 = pl.pallas_call(
        _tc1_body,
        grid=grid,
        in_specs=[_row_spec(DEG_W), _row_spec(DEG_W),
                  _row_spec(d_in), _full_spec((d_in, d_hid))],
        out_specs=[_row_spec(dhid), _row_spec(dhid)],
        out_shape=[jax.ShapeDtypeStruct((N_PAD, dhid), jnp.float32),
                   jax.ShapeDtypeStruct((N_PAD, dhid), jnp.float32)],
    )(dp0, dp1, x, W1)

    p = _sc_scatter(N_PAD, dhid, gs)(y10, y11, idx)

    y20, y21 = pl.pallas_call(
        _tc2_body,
        grid=grid,
        in_specs=[_row_spec(DEG_W), _row_spec(DEG_W),
                  _row_spec(dhid), _row_spec(dhid),
                  _row_spec(dhid), _row_spec(dhid),
                  _full_spec((1, d_hid)), _full_spec((d_hid, d_out))],
        out_specs=[_row_spec(dout), _row_spec(dout)],
        out_shape=[jax.ShapeDtypeStruct((N_PAD, dout), jnp.float32),
                   jax.ShapeDtypeStruct((N_PAD, dout), jnp.float32)],
    )(dp0, dp1, p[0, :n, :], p[1, :n, :], y10[:n], y11[:n],
      b1.reshape(1, d_hid), W2)

    q = _sc_scatter(N_PAD, dout, gs)(y20, y21, idx)

    out = pl.pallas_call(
        _tc3_body,
        grid=grid,
        in_specs=[_row_spec(DEG_W), _row_spec(DEG_W),
                  _row_spec(dout), _row_spec(dout),
                  _row_spec(dout), _row_spec(dout),
                  _full_spec((1, d_out))],
        out_specs=_row_spec(d_out),
        out_shape=jax.ShapeDtypeStruct((n, d_out), jnp.float32),
    )(dp0, dp1, q[0, :n, :], q[1, :n, :], y20[:n], y21[:n],
      b2.reshape(1, d_out))

    return out


# speedup vs baseline: 22.7063x; 1.2091x over previous
import functools

import jax
import jax.numpy as jnp
from jax import lax
from jax.experimental import pallas as pl
from jax.experimental.pallas import tpu as pltpu
from jax.experimental.pallas import tpu_sc as plsc

NC = 2
NS = 16
NW = NC * NS
CHUNK = 128
GB = 8
DEG_W = 8
N_PAD = 10240
ROW_BLK = 1000


def _sc_degree(n_pad, g):
    rpt = n_pad // NS
    nfill = rpt // CHUNK
    mesh = plsc.VectorSubcoreMesh(
        core_axis_name="c", subcore_axis_name="s",
        num_cores=NC, num_subcores=NS)

    def body(dsts, out, dbuf, rows, acc, sem):
        c = lax.axis_index("c")
        s = lax.axis_index("s")
        wid = s * NC + c
        base = s * rpt

        def fill(i, val):
            for j in range(DEG_W // 16):
                rows[i, pl.ds(j * 16, 16)] = jnp.full((16,), val, jnp.float32)
            return val

        lax.fori_loop(0, CHUNK, fill, 0.0)
        for k in range(nfill):
            pltpu.sync_copy(rows, acc.at[pl.ds(base + k * CHUNK, CHUNK)])
        lax.fori_loop(0, CHUNK, fill, 1.0)
        plsc.subcore_barrier()

        def step(gi, carry):
            pltpu.sync_copy(dsts.at[wid, gi], dbuf.at[0])
            pltpu.sync_copy(rows, acc.at[dbuf.at[0]], add=True)
            return carry

        lax.fori_loop(0, g, step, 0)
        plsc.subcore_barrier()
        for k in range(nfill):
            r0 = base + k * CHUNK
            pltpu.sync_copy(acc.at[pl.ds(r0, CHUNK)], rows)
            pltpu.sync_copy(rows, out.at[c, pl.ds(r0, CHUNK)])

    return pl.kernel(
        body,
        out_type=jax.ShapeDtypeStruct((NC, n_pad, DEG_W), jnp.float32),
        mesh=mesh,
        scratch_types=[
            pltpu.VMEM((1, CHUNK), jnp.int32),
            pltpu.VMEM((CHUNK, DEG_W), jnp.float32),
            pltpu.VMEM_SHARED((n_pad, DEG_W), jnp.float32),
            pltpu.SemaphoreType.DMA,
        ],
    )


def _sc_scatter(n_pad, dh, g):
    rpt = n_pad // NS
    nfill = rpt // CHUNK
    assert g % (2 * GB) == 0
    mesh = plsc.VectorSubcoreMesh(
        core_axis_name="c", subcore_axis_name="s",
        num_cores=NC, num_subcores=NS)

    def body(t0, t1, idx, out, iba, ibb, rows_a, rows_b, tsp, acc,
             sem_a, sem_b):
        c = lax.axis_index("c")
        s = lax.axis_index("s")
        base = s * rpt

        def fill(i, val):
            for j in range(dh // 16):
                rows_a[i, pl.ds(j * 16, 16)] = jnp.full((16,), val, jnp.float32)
            return val

        lax.fori_loop(0, CHUNK, fill, 0.0)
        for k in range(nfill):
            pltpu.sync_copy(rows_a, acc.at[pl.ds(base + k * CHUNK, CHUNK)])

        @pl.when(c == 0)
        def _():
            pltpu.sync_copy(t0.at[pl.ds(base, rpt)], tsp.at[pl.ds(base, rpt)])

        @pl.when(c == 1)
        def _():
            pltpu.sync_copy(t1.at[pl.ds(base, rpt)], tsp.at[pl.ds(base, rpt)])

        plsc.subcore_barrier()

        nb = g // GB
        pltpu.sync_copy(idx.at[s, pl.ds(0, GB)], iba)
        pltpu.async_copy(tsp.at[iba.at[0, 0]], rows_a, sem_a)

        rows = (rows_a, rows_b)
        sems = (sem_a, sem_b)

        def run_batch(bi, ib_this, ib_next):
            pltpu.sync_copy(
                idx.at[s, pl.ds((bi + 1) * GB, GB)], ib_next)
            for j in range(GB):
                t = j % 2
                nxt = ib_this.at[j + 1, 0] if j + 1 < GB else ib_next.at[0, 0]
                pltpu.async_copy(tsp.at[nxt], rows[1 - t], sems[1 - t])
                pltpu.make_async_copy(
                    tsp.at[ib_this.at[j, 0]], rows[t], sems[t]).wait()
                pltpu.sync_copy(rows[t], acc.at[ib_this.at[j, 1]], add=True)

        def step(i, carry):
            run_batch(2 * i, iba, ibb)
            run_batch(2 * i + 1, ibb, iba)
            return carry

        lax.fori_loop(0, nb // 2, step, 0)
        pltpu.make_async_copy(tsp.at[iba.at[0, 0]], rows_a, sem_a).wait()
        plsc.subcore_barrier()
        for k in range(nfill):
            r0 = base + k * CHUNK
            pltpu.sync_copy(acc.at[pl.ds(r0, CHUNK)], rows_a)
            pltpu.sync_copy(rows_a, out.at[c, pl.ds(r0, CHUNK)])

    return pl.kernel(
        body,
        out_type=jax.ShapeDtypeStruct((NC, n_pad, dh), jnp.float32),
        mesh=mesh,
        compiler_params=pltpu.CompilerParams(use_tc_tiling_on_sc=False),
        scratch_types=[
            pltpu.VMEM((GB, 2, CHUNK), jnp.int32),
            pltpu.VMEM((GB, 2, CHUNK), jnp.int32),
            pltpu.VMEM((CHUNK, dh), jnp.float32),
            pltpu.VMEM((CHUNK, dh), jnp.float32),
            pltpu.VMEM_SHARED((n_pad, dh), jnp.float32),
            pltpu.VMEM_SHARED((n_pad, dh), jnp.float32),
            pltpu.SemaphoreType.DMA,
            pltpu.SemaphoreType.DMA,
        ],
    )


def _deg_rsqrt(dp0, dp1):
    deg = dp0[:, 0:1] + dp1[:, 0:1] + 1.0
    return lax.rsqrt(deg)


def _tc1_body(dp0, dp1, x, w, o0, o1):
    d = _deg_rsqrt(dp0[...], dp1[...])
    y = d * jnp.dot(x[...], w[...], preferred_element_type=jnp.float32)
    dh = y.shape[1] // 2
    o0[...] = y[:, :dh]
    o1[...] = y[:, dh:]


def _tc2_body(dp0, dp1, p0, p1, y0, y1, b, w, o0, o1):
    d = _deg_rsqrt(dp0[...], dp1[...])
    s = jnp.concatenate([p0[...] + y0[...], p1[...] + y1[...]], axis=1)
    h = jnp.maximum(d * s + b[...], 0.0)
    y = d * jnp.dot(h, w[...], preferred_element_type=jnp.float32)
    dh = y.shape[1] // 2
    o0[...] = y[:, :dh]
    o1[...] = y[:, dh:]


def _tc3_body(dp0, dp1, q0, q1, y0, y1, b, o):
    d = _deg_rsqrt(dp0[...], dp1[...])
    z = jnp.concatenate([q0[...] + y0[...], q1[...] + y1[...]], axis=1)
    z = d * z + b[...]
    m = jnp.max(z, axis=1, keepdims=True)
    e = jnp.exp(z - m)
    o[...] = (z - m) - jnp.log(jnp.sum(e, axis=1, keepdims=True))


def _row_spec(width):
    return pl.BlockSpec((ROW_BLK, width), lambda i: (i, 0))


def _full_spec(shape):
    return pl.BlockSpec(shape, lambda i: tuple(0 for _ in shape))


def kernel(x, edge_index, W1, b1, W2, b2):
    n, d_in = x.shape
    d_hid = W1.shape[1]
    d_out = W2.shape[1]
    e = edge_index.shape[1]
    grid = (n // ROW_BLK,)

    ei = edge_index.astype(jnp.int32)

    gd = -(-e // (NW * CHUNK))
    dstd = jnp.full((NW * gd * CHUNK,), n, jnp.int32).at[:e].set(ei[1])
    dstd = dstd.reshape(NW, gd, CHUNK)

    gs = -(-e // (NS * CHUNK))
    gs = (gs + 2 * GB - 1) // (2 * GB) * (2 * GB)
    es = NS * gs * CHUNK
    src = jnp.full((es,), 0, jnp.int32).at[:e].set(ei[0]).reshape(NS, gs, CHUNK)
    dst = jnp.full((es,), n, jnp.int32).at[:e].set(ei[1]).reshape(NS, gs, CHUNK)
    idx = jnp.stack([src, dst], axis=2)
    pad = jnp.concatenate(
        [jnp.zeros((NS, GB, 1, CHUNK), jnp.int32),
         jnp.full((NS, GB, 1, CHUNK), n, jnp.int32)], axis=2)
    idx = jnp.concatenate([idx, pad], axis=1)

    dhid = d_hid // 2
    dout = d_out // 2

    degp = _sc_degree(N_PAD, gd)(dstd)
    dp0 = degp[0, :n, :]
    dp1 = degp[1, :n, :]

    y10, y11 = pl.pallas_call(
        _tc1_body,
        grid=grid,
        in_specs=[_row_spec(DEG_W), _row_spec(DEG_W),
                  _row_spec(d_in), _full_spec((d_in, d_hid))],
        out_specs=[_row_spec(dhid), _row_spec(dhid)],
        out_shape=[jax.ShapeDtypeStruct((N_PAD, dhid), jnp.float32),
                   jax.ShapeDtypeStruct((N_PAD, dhid), jnp.float32)],
    )(dp0, dp1, x, W1)

    p = _sc_scatter(N_PAD, dhid, gs)(y10, y11, idx)

    y20, y21 = pl.pallas_call(
        _tc2_body,
        grid=grid,
        in_specs=[_row_spec(DEG_W), _row_spec(DEG_W),
                  _row_spec(dhid), _row_spec(dhid),
                  _row_spec(dhid), _row_spec(dhid),
                  _full_spec((1, d_hid)), _full_spec((d_hid, d_out))],
        out_specs=[_row_spec(dout), _row_spec(dout)],
        out_shape=[jax.ShapeDtypeStruct((N_PAD, dout), jnp.float32),
                   jax.ShapeDtypeStruct((N_PAD, dout), jnp.float32)],
    )(dp0, dp1, p[0, :n, :], p[1, :n, :], y10[:n], y11[:n],
      b1.reshape(1, d_hid), W2)

    q = _sc_scatter(N_PAD, dout, gs)(y20, y21, idx)

    out = pl.pallas_call(
        _tc3_body,
        grid=grid,
        in_specs=[_row_spec(DEG_W), _row_spec(DEG_W),
                  _row_spec(dout), _row_spec(dout),
                  _row_spec(dout), _row_spec(dout),
                  _full_spec((1, d_out))],
        out_specs=_row_spec(d_out),
        out_shape=jax.ShapeDtypeStruct((n, d_out), jnp.float32),
    )(dp0, dp1, q[0, :n, :], q[1, :n, :], y20[:n], y21[:n],
      b2.reshape(1, d_out))

    return out


# speedup vs baseline: 25.8013x; 1.1363x over previous
import functools

import jax
import jax.numpy as jnp
from jax import lax
from jax.experimental import pallas as pl
from jax.experimental.pallas import tpu as pltpu
from jax.experimental.pallas import tpu_sc as plsc

NC = 2
NS = 16
NW = NC * NS
CHUNK = 128
GB = 8
DEG_W = 8
N_PAD = 10240
ROW_BLK = 1000


def _sc_degree(n_pad, g):
    rpt = n_pad // NS
    nfill = rpt // CHUNK
    mesh = plsc.VectorSubcoreMesh(
        core_axis_name="c", subcore_axis_name="s",
        num_cores=NC, num_subcores=NS)

    def body(dsts, out, dbuf, rows, acc, sem):
        c = lax.axis_index("c")
        s = lax.axis_index("s")
        wid = s * NC + c
        base = s * rpt

        def fill(i, val):
            for j in range(DEG_W // 16):
                rows[i, pl.ds(j * 16, 16)] = jnp.full((16,), val, jnp.float32)
            return val

        lax.fori_loop(0, CHUNK, fill, 0.0)
        for k in range(nfill):
            pltpu.sync_copy(rows, acc.at[pl.ds(base + k * CHUNK, CHUNK)])
        lax.fori_loop(0, CHUNK, fill, 1.0)
        plsc.subcore_barrier()

        def step(bi, carry):
            pltpu.sync_copy(dsts.at[wid, pl.ds(bi * GB, GB)], dbuf)
            for j in range(GB):
                pltpu.sync_copy(rows, acc.at[dbuf.at[j]], add=True)
            return carry

        lax.fori_loop(0, g // GB, step, 0)
        plsc.subcore_barrier()
        for k in range(nfill):
            r0 = base + k * CHUNK
            pltpu.sync_copy(acc.at[pl.ds(r0, CHUNK)], rows)
            pltpu.sync_copy(rows, out.at[c, pl.ds(r0, CHUNK)])

    return pl.kernel(
        body,
        out_type=jax.ShapeDtypeStruct((NC, n_pad, DEG_W), jnp.float32),
        mesh=mesh,
        scratch_types=[
            pltpu.VMEM((GB, CHUNK), jnp.int32),
            pltpu.VMEM((CHUNK, DEG_W), jnp.float32),
            pltpu.VMEM_SHARED((n_pad, DEG_W), jnp.float32),
            pltpu.SemaphoreType.DMA,
        ],
    )


def _sc_scatter(n_pad, dh, g):
    rpt = n_pad // NS
    nfill = rpt // CHUNK
    assert g % (2 * GB) == 0
    mesh = plsc.VectorSubcoreMesh(
        core_axis_name="c", subcore_axis_name="s",
        num_cores=NC, num_subcores=NS)

    def body(t0, t1, idx, out, iba, ibb, rows_a, rows_b, tsp, acc,
             sem_a, sem_b):
        c = lax.axis_index("c")
        s = lax.axis_index("s")
        base = s * rpt

        def fill(i, val):
            for j in range(dh // 16):
                rows_a[i, pl.ds(j * 16, 16)] = jnp.full((16,), val, jnp.float32)
            return val

        lax.fori_loop(0, CHUNK, fill, 0.0)
        for k in range(nfill):
            pltpu.sync_copy(rows_a, acc.at[pl.ds(base + k * CHUNK, CHUNK)])

        @pl.when(c == 0)
        def _():
            pltpu.sync_copy(t0.at[pl.ds(base, rpt)], tsp.at[pl.ds(base, rpt)])

        @pl.when(c == 1)
        def _():
            pltpu.sync_copy(t1.at[pl.ds(base, rpt)], tsp.at[pl.ds(base, rpt)])

        plsc.subcore_barrier()

        nb = g // GB
        pltpu.sync_copy(idx.at[s, pl.ds(0, GB)], iba)
        pltpu.async_copy(tsp.at[iba.at[0, 0]], rows_a, sem_a)

        rows = (rows_a, rows_b)
        sems = (sem_a, sem_b)

        def run_batch(bi, ib_this, ib_next):
            pltpu.sync_copy(
                idx.at[s, pl.ds((bi + 1) * GB, GB)], ib_next)
            for j in range(GB):
                t = j % 2
                nxt = ib_this.at[j + 1, 0] if j + 1 < GB else ib_next.at[0, 0]
                pltpu.async_copy(tsp.at[nxt], rows[1 - t], sems[1 - t])
                pltpu.make_async_copy(
                    tsp.at[ib_this.at[j, 0]], rows[t], sems[t]).wait()
                pltpu.sync_copy(rows[t], acc.at[ib_this.at[j, 1]], add=True)

        def step(i, carry):
            run_batch(2 * i, iba, ibb)
            run_batch(2 * i + 1, ibb, iba)
            return carry

        lax.fori_loop(0, nb // 2, step, 0)
        pltpu.make_async_copy(tsp.at[iba.at[0, 0]], rows_a, sem_a).wait()
        plsc.subcore_barrier()
        for k in range(nfill):
            r0 = base + k * CHUNK
            pltpu.sync_copy(acc.at[pl.ds(r0, CHUNK)], rows_a)
            pltpu.sync_copy(rows_a, out.at[c, pl.ds(r0, CHUNK)])

    return pl.kernel(
        body,
        out_type=jax.ShapeDtypeStruct((NC, n_pad, dh), jnp.float32),
        mesh=mesh,
        compiler_params=pltpu.CompilerParams(use_tc_tiling_on_sc=False),
        scratch_types=[
            pltpu.VMEM((GB, 2, CHUNK), jnp.int32),
            pltpu.VMEM((GB, 2, CHUNK), jnp.int32),
            pltpu.VMEM((CHUNK, dh), jnp.float32),
            pltpu.VMEM((CHUNK, dh), jnp.float32),
            pltpu.VMEM_SHARED((n_pad, dh), jnp.float32),
            pltpu.VMEM_SHARED((n_pad, dh), jnp.float32),
            pltpu.SemaphoreType.DMA,
            pltpu.SemaphoreType.DMA,
        ],
    )


def _deg_rsqrt(dp):
    deg = dp[0, :, 0:1] + dp[1, :, 0:1] + 1.0
    return lax.rsqrt(deg)


def _tc1_body(dp, x, w, o0, o1):
    d = _deg_rsqrt(dp[...])
    y = d * jnp.dot(x[...], w[...], preferred_element_type=jnp.float32)
    dh = y.shape[1] // 2
    o0[...] = y[:, :dh]
    o1[...] = y[:, dh:]


def _tc2_body(dp, p, y0, y1, b, w, o0, o1):
    d = _deg_rsqrt(dp[...])
    s = jnp.concatenate([p[0] + y0[...], p[1] + y1[...]], axis=1)
    h = jnp.maximum(d * s + b[...], 0.0)
    y = d * jnp.dot(h, w[...], preferred_element_type=jnp.float32)
    dh = y.shape[1] // 2
    o0[...] = y[:, :dh]
    o1[...] = y[:, dh:]


def _tc3_body(dp, q, y0, y1, b, o):
    d = _deg_rsqrt(dp[...])
    z = jnp.concatenate([q[0] + y0[...], q[1] + y1[...]], axis=1)
    z = d * z + b[...]
    m = jnp.max(z, axis=1, keepdims=True)
    e = jnp.exp(z - m)
    o[...] = (z - m) - jnp.log(jnp.sum(e, axis=1, keepdims=True))


def _row_spec(width):
    return pl.BlockSpec((ROW_BLK, width), lambda i: (i, 0))


def _pair_spec(width):
    return pl.BlockSpec((2, ROW_BLK, width), lambda i: (0, i, 0))


def _full_spec(shape):
    return pl.BlockSpec(shape, lambda i: tuple(0 for _ in shape))


def kernel(x, edge_index, W1, b1, W2, b2):
    n, d_in = x.shape
    d_hid = W1.shape[1]
    d_out = W2.shape[1]
    e = edge_index.shape[1]
    grid = (n // ROW_BLK,)

    ei = edge_index.astype(jnp.int32)

    gd = -(-e // (NW * CHUNK))
    gd = (gd + GB - 1) // GB * GB
    dstd = jnp.full((NW * gd * CHUNK,), n, jnp.int32).at[:e].set(ei[1])
    dstd = dstd.reshape(NW, gd, CHUNK)

    gs = -(-e // (NS * CHUNK))
    gs = (gs + 2 * GB - 1) // (2 * GB) * (2 * GB)
    es = NS * gs * CHUNK
    src = jnp.full((es,), 0, jnp.int32).at[:e].set(ei[0]).reshape(NS, gs, CHUNK)
    dst = jnp.full((es,), n, jnp.int32).at[:e].set(ei[1]).reshape(NS, gs, CHUNK)
    idx = jnp.stack([src, dst], axis=2)
    pad = jnp.concatenate(
        [jnp.zeros((NS, GB, 1, CHUNK), jnp.int32),
         jnp.full((NS, GB, 1, CHUNK), n, jnp.int32)], axis=2)
    idx = jnp.concatenate([idx, pad], axis=1)

    dhid = d_hid // 2
    dout = d_out // 2

    degp = _sc_degree(N_PAD, gd)(dstd)

    y10, y11 = pl.pallas_call(
        _tc1_body,
        grid=grid,
        in_specs=[_pair_spec(DEG_W),
                  _row_spec(d_in), _full_spec((d_in, d_hid))],
        out_specs=[_row_spec(dhid), _row_spec(dhid)],
        out_shape=[jax.ShapeDtypeStruct((N_PAD, dhid), jnp.float32),
                   jax.ShapeDtypeStruct((N_PAD, dhid), jnp.float32)],
    )(degp, x, W1)

    p = _sc_scatter(N_PAD, dhid, gs)(y10, y11, idx)

    y20, y21 = pl.pallas_call(
        _tc2_body,
        grid=grid,
        in_specs=[_pair_spec(DEG_W), _pair_spec(dhid),
                  _row_spec(dhid), _row_spec(dhid),
                  _full_spec((1, d_hid)), _full_spec((d_hid, d_out))],
        out_specs=[_row_spec(dout), _row_spec(dout)],
        out_shape=[jax.ShapeDtypeStruct((N_PAD, dout), jnp.float32),
                   jax.ShapeDtypeStruct((N_PAD, dout), jnp.float32)],
    )(degp, p, y10, y11, b1.reshape(1, d_hid), W2)

    q = _sc_scatter(N_PAD, dout, gs)(y20, y21, idx)

    out = pl.pallas_call(
        _tc3_body,
        grid=grid,
        in_specs=[_pair_spec(DEG_W), _pair_spec(dout),
                  _row_spec(dout), _row_spec(dout),
                  _full_spec((1, d_out))],
        out_specs=_row_spec(d_out),
        out_shape=jax.ShapeDtypeStruct((n, d_out), jnp.float32),
    )(degp, q, y20, y21, b2.reshape(1, d_out))

    return out
